# Initial kernel scaffold; baseline (speedup 1.0000x reference)
import jax
import jax.numpy as jnp
from jax.experimental import pallas as pl


def kernel(actions, table):
    raise NotImplementedError("write your pallas kernel here")


# speedup vs baseline: 2.5701x; 2.5701x over previous
import functools

import jax
import jax.numpy as jnp
from jax import lax
from jax.experimental import pallas as pl
from jax.experimental.pallas import tpu as pltpu
from jax.experimental.pallas import tpu_sc as plsc

_INFO = plsc.get_sparse_core_info()
_NC, _NS = _INFO.num_cores, _INFO.num_subcores
_NW = _NC * _NS
_L = 16

_CHUNK = 1024
_PAIRS = _CHUNK // 2
_GATHER_ROW = 128


@functools.partial(jax.jit, static_argnames=("n", "d"))
def _emb_lookup(tp, idx1d, *, n, d):
    per_w = n // _NW
    n_chunks = per_w // _CHUNK
    mesh = plsc.VectorSubcoreMesh(core_axis_name="c", subcore_axis_name="s")

    @functools.partial(
        pl.kernel,
        mesh=mesh,
        out_type=jax.ShapeDtypeStruct((n // 2, 2 * d), jnp.float32),
        scratch_types=[
            pltpu.VMEM((_CHUNK,), jnp.int32),
            pltpu.VMEM((_PAIRS,), jnp.int32),
            pltpu.VMEM((_PAIRS, 2 * d), jnp.float32),
            pltpu.SemaphoreType.DMA,
        ],
    )
    def k(tp_hbm, idx_hbm, out_hbm, idx_v, pid_v, rows_v, sem):
        wid = lax.axis_index("s") * _NC + lax.axis_index("c")
        idx0 = wid * per_w
        pair0 = idx0 // 2
        lane = lax.iota(jnp.int32, _L)
        low_half = lane < 8
        ev_sel = jnp.arange(0, 2 * _L, 2, dtype=jnp.int32) % _L
        od_sel = ev_sel + 1

        def deinterleave(v, sel):
            return v.at[sel].get(mode="promise_in_bounds")

        def body(i, carry):
            ib = pl.multiple_of(idx0 + i * _CHUNK, _CHUNK)
            pb = pl.multiple_of(pair0 + i * _PAIRS, _PAIRS)
            pltpu.sync_copy(idx_hbm.at[pl.ds(ib, _CHUNK)], idx_v)
            for g in range(_PAIRS // _L):
                v0 = idx_v[pl.ds(2 * _L * g, _L)]
                v1 = idx_v[pl.ds(2 * _L * g + _L, _L)]
                ev = jnp.where(low_half, deinterleave(v0, ev_sel),
                               deinterleave(v1, ev_sel))
                od = jnp.where(low_half, deinterleave(v0, od_sel),
                               deinterleave(v1, od_sel))
                pid_v[pl.ds(g * _L, _L)] = ev * 8 + od
            copies = []
            for q in range(_PAIRS // _GATHER_ROW):
                copies.append(pltpu.async_copy(
                    tp_hbm.at[pid_v.at[pl.ds(q * _GATHER_ROW, _GATHER_ROW)]],
                    rows_v.at[pl.ds(q * _GATHER_ROW, _GATHER_ROW)],
                    sem))
            for c in copies:
                c.wait()
            pltpu.sync_copy(rows_v, out_hbm.at[pl.ds(pb, _PAIRS)])
            return carry

        lax.fori_loop(0, n_chunks, body, 0)

    return k(tp, idx1d)


def kernel(actions, table):
    B, T, Hp, Wp = actions.shape
    n = B * T * Hp * Wp
    d = table.shape[-1]
    v = table.shape[0]
    idx1d = actions.reshape(n).astype(jnp.int32)
    tp = jnp.concatenate(
        [jnp.repeat(table, v, axis=0), jnp.tile(table, (v, 1))], axis=1)
    out2 = _emb_lookup(tp, idx1d, n=n, d=d)
    return out2.reshape(B, T, Hp, Wp, d)


# speedup vs baseline: 2.5796x; 1.0037x over previous
import functools

import jax
import jax.numpy as jnp
from jax import lax
from jax.experimental import pallas as pl
from jax.experimental.pallas import tpu as pltpu
from jax.experimental.pallas import tpu_sc as plsc

_INFO = plsc.get_sparse_core_info()
_NC, _NS = _INFO.num_cores, _INFO.num_subcores
_NW = _NC * _NS
_L = 16

_CHUNK = 256
_PAIRS = _CHUNK // 2
_NBUF = 4


@functools.partial(jax.jit, static_argnames=("n", "d"))
def _emb_lookup(tp, idx1d, *, n, d):
    per_w = n // _NW
    n_chunks = per_w // _CHUNK
    assert n_chunks % _NBUF == 0 and n_chunks > 2 * _NBUF
    mesh = plsc.VectorSubcoreMesh(core_axis_name="c", subcore_axis_name="s")

    @functools.partial(
        pl.kernel,
        mesh=mesh,
        out_type=jax.ShapeDtypeStruct((n // 2, 2 * d), jnp.float32),
        scratch_types=[
            pltpu.VMEM((per_w,), jnp.int32),
            pltpu.VMEM((_NBUF, _PAIRS), jnp.int32),
            pltpu.VMEM((_NBUF, _PAIRS, 2 * d), jnp.float32),
            pltpu.SemaphoreType.DMA,
            pltpu.SemaphoreType.DMA,
            pltpu.SemaphoreType.DMA,
            pltpu.SemaphoreType.DMA,
            pltpu.SemaphoreType.DMA,
            pltpu.SemaphoreType.DMA,
            pltpu.SemaphoreType.DMA,
            pltpu.SemaphoreType.DMA,
        ],
    )
    def k(tp_hbm, idx_hbm, out_hbm, idx_all, pid_v, rows_v, *sems):
        sem_g = sems[:_NBUF]
        sem_w = sems[_NBUF:]
        wid = lax.axis_index("s") * _NC + lax.axis_index("c")
        idx0 = wid * per_w
        pair0 = idx0 // 2
        lane = lax.iota(jnp.int32, _L)
        low_half = lane < 8
        ev_sel = jnp.arange(0, 2 * _L, 2, dtype=jnp.int32) % _L
        od_sel = ev_sel + 1

        def deinterleave(v, sel):
            return v.at[sel].get(mode="promise_in_bounds")

        def compute_pids(j, b):
            base = j * _CHUNK
            for g in range(_PAIRS // _L):
                v0 = idx_all[pl.ds(base + 2 * _L * g, _L)]
                v1 = idx_all[pl.ds(base + 2 * _L * g + _L, _L)]
                ev = jnp.where(low_half, deinterleave(v0, ev_sel),
                               deinterleave(v1, ev_sel))
                od = jnp.where(low_half, deinterleave(v0, od_sel),
                               deinterleave(v1, od_sel))
                pid_v[b, pl.ds(g * _L, _L)] = ev * 8 + od

        def fire_gather(j, b):
            compute_pids(j, b)
            pltpu.make_async_copy(
                tp_hbm.at[pid_v.at[b]], rows_v.at[b], sem_g[b]).start()

        def wait_gather(b):
            pltpu.make_async_copy(
                out_hbm.at[pl.ds(0, _PAIRS)], rows_v.at[b], sem_g[b]).wait()

        def fire_write(j, b):
            pb = pl.multiple_of(pair0 + j * _PAIRS, _PAIRS)
            pltpu.make_async_copy(
                rows_v.at[b], out_hbm.at[pl.ds(pb, _PAIRS)], sem_w[b]).start()

        def drain_write(b):
            pltpu.make_async_copy(
                rows_v.at[b], out_hbm.at[pl.ds(0, _PAIRS)], sem_w[b]).wait()

        ib = pl.multiple_of(idx0, _CHUNK)
        pltpu.sync_copy(idx_hbm.at[pl.ds(ib, per_w)], idx_all)

        fire_gather(0, 0)
        for b in range(1, _NBUF):
            fire_gather(b, b)
            wait_gather(b - 1)
            fire_write(b - 1, b - 1)

        def body(k, carry):
            for b in range(_NBUF):
                j = k * _NBUF + b
                drain_write(b)
                fire_gather(j, b)
                prev = (b - 1) % _NBUF
                wait_gather(prev)
                fire_write(j - 1, prev)
            return carry

        lax.fori_loop(1, n_chunks // _NBUF, body, 0)

        last = _NBUF - 1
        wait_gather(last)
        fire_write(n_chunks - 1, last)
        for b in range(_NBUF):
            drain_write(b)

    return k(tp, idx1d)


def kernel(actions, table):
    B, T, Hp, Wp = actions.shape
    n = B * T * Hp * Wp
    d = table.shape[-1]
    v = table.shape[0]
    idx1d = actions.reshape(n).astype(jnp.int32)
    tp = jnp.concatenate(
        [jnp.repeat(table, v, axis=0), jnp.tile(table, (v, 1))], axis=1)
    out2 = _emb_lookup(tp, idx1d, n=n, d=d)
    return out2.reshape(B, T, Hp, Wp, d)
